# baseline (speedup 1.0000x reference)
import jax
import jax.numpy as jnp
from jax import lax
from jax.experimental import pallas as pl
from jax.experimental.pallas import tpu as pltpu
from jax.experimental.pallas import tpu_sc as plsc

S_BLK = 512
F_BLK = 2048

SC_B = 2
FW = 64 * SC_B
NG = FW // 16
R = 128

_MESH = plsc.VectorSubcoreMesh(core_axis_name="core", subcore_axis_name="subcore")


def _tc_body(x_ref, l_ref, o_ref, carry_ref):
    s = pl.program_id(2)

    @pl.when(s == 0)
    def _():
        carry_ref[...] = jnp.zeros_like(carry_ref)

    x = x_ref[0]
    hi = x.astype(jnp.bfloat16)
    ltri = l_ref[...]
    e = jnp.dot(ltri, hi, preferred_element_type=jnp.float32)
    c = carry_ref[...]
    o_ref[0] = e + c
    carry_ref[...] = c + jnp.sum(x, axis=0, keepdims=True)


def _tc_half(x):
    b, s, f = x.shape
    ns = s // S_BLK
    nf = f // F_BLK
    row = jax.lax.broadcasted_iota(jnp.int32, (S_BLK, S_BLK), 0)
    col = jax.lax.broadcasted_iota(jnp.int32, (S_BLK, S_BLK), 1)
    ltri = (col < row).astype(jnp.bfloat16)
    return pl.pallas_call(
        _tc_body,
        grid=(b, nf, ns),
        in_specs=[
            pl.BlockSpec((1, S_BLK, F_BLK), lambda b, jf, js: (b, js, jf)),
            pl.BlockSpec((S_BLK, S_BLK), lambda b, jf, js: (0, 0)),
        ],
        out_specs=pl.BlockSpec((1, S_BLK, F_BLK), lambda b, jf, js: (b, js, jf)),
        out_shape=jax.ShapeDtypeStruct((b, s, f), x.dtype),
        scratch_shapes=[pltpu.VMEM((1, F_BLK), x.dtype)],
        compiler_params=pltpu.CompilerParams(
            dimension_semantics=("parallel", "parallel", "arbitrary"),
        ),
    )(x, ltri)


def _sc_half(x):
    b, s, f = x.shape
    n_chunks = s // R
    n_slabs = f // FW
    n_tasks = n_slabs * b

    @pl.kernel(
        out_type=jax.ShapeDtypeStruct((b, s, f), x.dtype),
        mesh=_MESH,
        scratch_types=[pltpu.VMEM((FW,), x.dtype)],
    )
    def sc_cumsum(x_hbm, o_hbm, carry_ref):
        for g in range(NG):
            carry_ref[pl.ds(16 * g, 16)] = jnp.zeros((16,), x.dtype)

        def body(x_vmem, o_vmem):
            def row_step(r, carry):
                new = []
                for g in range(NG):
                    cg = carry[g]
                    o_vmem[0, r, pl.ds(16 * g, 16)] = cg
                    new.append(cg + x_vmem[0, r, pl.ds(16 * g, 16)])
                return tuple(new)

            c0 = tuple(carry_ref[pl.ds(16 * g, 16)] for g in range(NG))
            cn = lax.fori_loop(0, R, row_step, c0, unroll=False)
            for g in range(NG):
                carry_ref[pl.ds(16 * g, 16)] = cn[g]

        pltpu.emit_pipeline(
            body,
            grid=(n_tasks, n_chunks),
            in_specs=[
                pl.BlockSpec(
                    (1, R, FW),
                    index_map=lambda t, k: (t // n_slabs, k, t % n_slabs),
                ),
            ],
            out_specs=[
                pl.BlockSpec(
                    (1, R, FW),
                    index_map=lambda t, k: (t // n_slabs, k, t % n_slabs),
                ),
            ],
            core_axis_name=("core", "subcore"),
            dimension_semantics=(pltpu.PARALLEL, pltpu.ARBITRARY),
        )(x_hbm, o_hbm)

    return sc_cumsum(x)


@jax.jit
def kernel(x):
    b = x.shape[0]
    tc_b = b - SC_B
    out_tc = _tc_half(x[:tc_b])
    out_sc = _sc_half(x[tc_b:])
    return jnp.concatenate([out_tc, out_sc], axis=0)


# speedup vs baseline: 2.4233x; 2.4233x over previous
import jax
import jax.numpy as jnp
from jax import lax
from jax.experimental import pallas as pl
from jax.experimental.pallas import tpu as pltpu
from jax.experimental.pallas import tpu_sc as plsc

FW = 256
NG = FW // 16
R = 64

_MESH = plsc.VectorSubcoreMesh(core_axis_name="core", subcore_axis_name="subcore")


@jax.jit
def kernel(x):
    b, s, f = x.shape
    n_chunks = s // R
    n_tasks = (f // FW) * b

    @pl.kernel(
        out_type=jax.ShapeDtypeStruct((b, s, f), x.dtype),
        mesh=_MESH,
        scratch_types=[pltpu.VMEM((FW,), x.dtype)],
    )
    def sc_cumsum(x_hbm, o_hbm, carry_ref):
        for g in range(NG):
            carry_ref[pl.ds(16 * g, 16)] = jnp.zeros((16,), x.dtype)

        def body(x_vmem, o_vmem):
            def row(r, carry):
                new = []
                for g in range(NG):
                    cg = carry[g]
                    o_vmem[0, r, pl.ds(16 * g, 16)] = cg
                    new.append(cg + x_vmem[0, r, pl.ds(16 * g, 16)])
                return tuple(new)

            c0 = tuple(carry_ref[pl.ds(16 * g, 16)] for g in range(NG))
            cn = lax.fori_loop(0, R, row, c0, unroll=4)
            for g in range(NG):
                carry_ref[pl.ds(16 * g, 16)] = cn[g]

        pltpu.emit_pipeline(
            body,
            grid=(n_tasks, n_chunks),
            in_specs=[
                pl.BlockSpec((1, R, FW), index_map=lambda t, k: (t // 8, k, t % 8)),
            ],
            out_specs=[
                pl.BlockSpec((1, R, FW), index_map=lambda t, k: (t // 8, k, t % 8)),
            ],
            core_axis_name=("core", "subcore"),
            dimension_semantics=(pltpu.PARALLEL, pltpu.ARBITRARY),
        )(x_hbm, o_hbm)

    return sc_cumsum(x)
